# baseline (speedup 1.0000x reference)
import functools

import jax
import jax.numpy as jnp
from jax import lax
from jax.experimental import pallas as pl
from jax.experimental.pallas import tpu as pltpu
from jax.experimental.pallas import tpu_sc as plsc

_NC = 2
_NS = 16
_NW = _NC * _NS
_LANES = 16

_BLK = 128

_NEG = -1e30


def _mesh():
    return plsc.VectorSubcoreMesh(
        core_axis_name="c", subcore_axis_name="s",
        num_cores=_NC, num_subcores=_NS)


def _wid():
    return lax.axis_index("s") * _NC + lax.axis_index("c")


def _logits_body(x_ref, rw_ref, lg_ref):
    lg_ref[...] = jax.lax.dot_general(
        rw_ref[...], x_ref[...], (((1,), (1,)), ((), ())))


def _route_body(T, E, lg_hbm, e_hbm, w_hbm, cnt_hbm,
                lg_v, e_v, w_v, cnt_v, sem):
    tt = T // _NW
    wid = _wid()
    t0 = wid * tt
    for e in range(E):
        pltpu.async_copy(lg_hbm.at[e, pl.ds(t0, tt)], lg_v.at[e], sem).wait()

    iota = lax.iota(jnp.int32, _LANES)
    cnt_acc = [jnp.zeros((_LANES,), jnp.int32) for _ in range(E)]
    for g in range(tt // _LANES):
        lv = [lg_v[e, pl.ds(g * _LANES, _LANES)] for e in range(E)]
        m1 = lv[0]
        for e in range(1, E):
            m1 = jnp.maximum(m1, lv[e])
        i1 = jnp.full((_LANES,), E, jnp.int32)
        for e in reversed(range(E)):
            i1 = jnp.where(lv[e] == m1, e, i1)
        m2 = jnp.full((_LANES,), _NEG)
        for e in range(E):
            le = jnp.where(i1 == e, _NEG, lv[e])
            m2 = jnp.maximum(m2, le)
        i2 = jnp.full((_LANES,), E, jnp.int32)
        for e in reversed(range(E)):
            i2 = jnp.where((lv[e] == m2) & (i1 != e), e, i2)
        s2 = jnp.exp(m2 - m1)
        w1v = 1.0 / (1.0 + s2)
        w2v = 1.0 - w1v
        e_v[0, pl.ds(g * _LANES, _LANES)] = i1
        e_v[1, pl.ds(g * _LANES, _LANES)] = i2
        w_v[0, pl.ds(g * _LANES, _LANES)] = w1v
        w_v[1, pl.ds(g * _LANES, _LANES)] = w2v
        for e in range(E):
            c1 = plsc.all_reduce_population_count(i1 == e)
            c2 = plsc.all_reduce_population_count(i2 == e)
            cnt_acc[e] = cnt_acc[e] + c1 + c2

    row = jnp.zeros((_LANES,), jnp.int32)
    for e in range(E):
        row = jnp.where(iota == e, cnt_acc[e], row)
    cnt_v[...] = row

    for j in range(2):
        pltpu.async_copy(e_v.at[j], e_hbm.at[j, pl.ds(t0, tt)], sem).wait()
        pltpu.async_copy(w_v.at[j], w_hbm.at[j, pl.ds(t0, tt)], sem).wait()
    pltpu.async_copy(cnt_v, cnt_hbm.at[wid], sem).wait()


def _dispatch_body(T, E, NB, e_hbm, w_hbm, cnt_hbm, x_hbm,
                   xs_hbm, ws_hbm, pos_hbm, be_hbm, nb_hbm,
                   cnt_all, e_v, w_v, pos_v, posbuf, rows_v, wbufs, bebuf,
                   nbbuf, sem):
    tt = T // _NW
    na = 2 * tt
    nchunk = na // 32
    wid = _wid()
    t0 = wid * tt

    pltpu.async_copy(cnt_hbm, cnt_all, sem).wait()
    for j in range(2):
        pltpu.async_copy(e_hbm.at[j, pl.ds(t0, tt)], e_v.at[j], sem).wait()
        pltpu.async_copy(w_hbm.at[j, pl.ds(t0, tt)], w_v.at[j], sem).wait()

    widv = jnp.full((_LANES,), wid, jnp.int32)
    tot = jnp.zeros((_LANES,), jnp.int32)
    pref = jnp.zeros((_LANES,), jnp.int32)
    for s in range(_NW):
        rowc = cnt_all[s, :]
        tot = tot + rowc
        sv = jnp.full((_LANES,), s, jnp.int32)
        pref = pref + jnp.where(sv < widv, rowc, 0)
    padded = ((tot + (_BLK - 1)) >> 7) << 7
    offs_incl = plsc.cumsum(padded)
    offs_excl = offs_incl - padded
    mybase = offs_excl + pref

    carry = [jnp.int32(0) for _ in range(E)]
    for g in range(na // _LANES):
        j, gg = g // (tt // _LANES), g % (tt // _LANES)
        ev = e_v[j, pl.ds(gg * _LANES, _LANES)]
        pos_g = jnp.zeros((_LANES,), jnp.int32)
        for e in range(E):
            m = ev == e
            mi = m.astype(jnp.int32)
            cs = plsc.cumsum(mi)
            rank = (cs - mi) + carry[e]
            pos_g = jnp.where(m, mybase[e] + rank, pos_g)
            carry[e] = carry[e] + cs[_LANES - 1]
        pos_v[j, pl.ds(gg * _LANES, _LANES)] = pos_g
        posbuf[g >> 1, pl.ds((g & 1) * _LANES, _LANES)] = pos_g

    for j in range(2):
        pltpu.async_copy(pos_v.at[j], pos_hbm.at[j, pl.ds(t0, tt)], sem).wait()

    for c in range(nchunk):
        j = c // (nchunk // 2)
        base = (c & 1) * 32
        wva = w_v[j, pl.ds(base, _LANES)]
        wvb = w_v[j, pl.ds(base + _LANES, _LANES)]
        for i in range(32):
            w_s = wva[i] if i < _LANES else wvb[i - _LANES]
            wbufs[c, i, pl.ds(0, _LANES)] = jnp.full((_LANES,), w_s)

    pltpu.async_copy(x_hbm.at[pl.ds(t0, tt)], rows_v, sem).wait()
    copies = []
    for c in range(nchunk):
        src = rows_v.at[pl.ds((c & 1) * 32, 32)]
        copies.append(pltpu.async_copy(src, xs_hbm.at[posbuf.at[c]], sem))
        copies.append(
            pltpu.async_copy(wbufs.at[c], ws_hbm.at[posbuf.at[c]], sem))
    for cp in copies:
        cp.wait()

    @pl.when(wid == 0)
    def _emit_block_map():
        iota = lax.iota(jnp.int32, _LANES)
        for bg in range(NB // _LANES):
            rowb = (iota + bg * _LANES) * _BLK
            acc = jnp.zeros((_LANES,), jnp.int32)
            for e in range(E):
                acc = acc + jnp.where(rowb >= offs_incl[e], 1, 0)
            bebuf[pl.ds(bg * _LANES, _LANES)] = jnp.minimum(acc, E - 1)
        nbbuf[...] = jnp.full((_LANES,), offs_incl[E - 1] >> 7, jnp.int32)
        pltpu.async_copy(bebuf, be_hbm, sem).wait()
        pltpu.async_copy(nbbuf, nb_hbm, sem).wait()


def _gemm_body(be_ref, nb_ref, x_ref, ws_ref, w1_ref, v1_ref, w2_ref, y_ref):
    i = pl.program_id(0)

    @pl.when(i < nb_ref[0])
    def _():
        x = x_ref[...]
        dn = (((1,), (1,)), ((), ()))
        h1 = jax.lax.dot_general(
            x, w1_ref[0], dn, preferred_element_type=jnp.float32)
        h2 = jax.lax.dot_general(
            x, v1_ref[0], dn, preferred_element_type=jnp.float32)
        g = ((h1 * jax.nn.sigmoid(h1)) * h2).astype(jnp.bfloat16)
        y = jax.lax.dot_general(
            g, w2_ref[0], (((1,), (0,)), ((), ())),
            preferred_element_type=jnp.float32)
        y_ref[...] = y * ws_ref[:, 0:1]


def _combine_body(T, H, y_hbm, pos_hbm, out_hbm,
                  pos_v, r0_v, r1_v, out_v, sem0, sem1, sem2):
    tt = T // _NW
    nchunk = tt // _LANES
    wid = _wid()
    t0 = wid * tt
    pltpu.async_copy(pos_hbm.at[0, pl.ds(t0, tt)], pos_v.at[0], sem0).wait()
    pltpu.async_copy(pos_hbm.at[1, pl.ds(t0, tt)], pos_v.at[1], sem0).wait()

    def fire(c, sem):
        idx0 = pos_v[0, pl.ds(c * _LANES, _LANES)]
        idx1 = pos_v[1, pl.ds(c * _LANES, _LANES)]
        b = c & 1
        cp0 = pltpu.async_copy(y_hbm.at[idx0], r0_v.at[b], sem)
        cp1 = pltpu.async_copy(y_hbm.at[idx1], r1_v.at[b], sem)
        return cp0, cp1

    pend = fire(0, sem0)
    for c in range(nchunk):
        nxt = fire(c + 1, sem1 if (c & 1) == 0 else sem0) \
            if c + 1 < nchunk else None
        for cp in pend:
            cp.wait()
        b = c & 1
        u = 4
        for tk in range(_LANES):
            def h_body(h, _, tk=tk, b=b):
                for uu in range(u):
                    off = pl.multiple_of(h * _LANES * u + uu * _LANES,
                                         _LANES)
                    out_v[tk, pl.ds(off, _LANES)] = (
                        r0_v[b, tk, pl.ds(off, _LANES)]
                        + r1_v[b, tk, pl.ds(off, _LANES)])
                return 0

            lax.fori_loop(0, H // (_LANES * u), h_body, 0)
        pltpu.async_copy(
            out_v, out_hbm.at[pl.ds(t0 + c * _LANES, _LANES)], sem2).wait()
        pend = nxt


def kernel(hidden_states, router_w, w1, v1, w2):
    B, S, H = hidden_states.shape
    E, F, _ = w1.shape
    x = jnp.transpose(hidden_states, (1, 0, 2)).reshape(-1, H)
    T = x.shape[0]
    assert T % (_NW * _LANES) == 0

    NB = 2 * T // _BLK + E
    NB = (NB + _LANES - 1) // _LANES * _LANES
    NPAD = NB * _BLK
    tt = T // _NW

    logits = pl.pallas_call(
        _logits_body,
        grid=(1,),
        in_specs=[
            pl.BlockSpec((T, H), lambda i: (0, 0)),
            pl.BlockSpec((E, H), lambda i: (0, 0)),
        ],
        out_specs=pl.BlockSpec((E, T), lambda i: (0, 0)),
        out_shape=jax.ShapeDtypeStruct((E, T), jnp.float32),
    )(x, router_w)

    route = pl.kernel(
        functools.partial(_route_body, T, E),
        out_type=(
            jax.ShapeDtypeStruct((2, T), jnp.int32),
            jax.ShapeDtypeStruct((2, T), jnp.float32),
            jax.ShapeDtypeStruct((_NW, _LANES), jnp.int32),
        ),
        mesh=_mesh(),
        compiler_params=pltpu.CompilerParams(needs_layout_passes=False),
        scratch_types=[
            pltpu.VMEM((E, tt), jnp.float32),
            pltpu.VMEM((2, tt), jnp.int32),
            pltpu.VMEM((2, tt), jnp.float32),
            pltpu.VMEM((_LANES,), jnp.int32),
            pltpu.SemaphoreType.DMA,
        ],
    )
    e_top, w_top, cnt = route(logits)

    x16p = jax.lax.bitcast_convert_type(
        x.astype(jnp.bfloat16).reshape(T, H // 2, 2), jnp.int32)
    dispatch = pl.kernel(
        functools.partial(_dispatch_body, T, E, NB),
        out_type=(
            jax.ShapeDtypeStruct((NPAD, H // 2), jnp.int32),
            jax.ShapeDtypeStruct((NPAD, 128), jnp.float32),
            jax.ShapeDtypeStruct((2, T), jnp.int32),
            jax.ShapeDtypeStruct((NB,), jnp.int32),
            jax.ShapeDtypeStruct((_LANES,), jnp.int32),
        ),
        mesh=_mesh(),
        compiler_params=pltpu.CompilerParams(needs_layout_passes=False),
        scratch_types=[
            pltpu.VMEM((_NW, _LANES), jnp.int32),
            pltpu.VMEM((2, tt), jnp.int32),
            pltpu.VMEM((2, tt), jnp.float32),
            pltpu.VMEM((2, tt), jnp.int32),
            pltpu.VMEM((2 * tt // 32, 32), jnp.int32),
            pltpu.VMEM((tt, H // 2), jnp.int32),
            pltpu.VMEM((2 * tt // 32, 32, 128), jnp.float32),
            pltpu.VMEM((NB,), jnp.int32),
            pltpu.VMEM((_LANES,), jnp.int32),
            pltpu.SemaphoreType.DMA,
        ],
    )
    x_sorted_p, ws, pos, be, nb_used = dispatch(e_top, w_top, cnt, x16p)
    x_sorted = jax.lax.bitcast_convert_type(
        x_sorted_p, jnp.bfloat16).reshape(NPAD, H)

    y_sorted = pl.pallas_call(
        _gemm_body,
        grid_spec=pltpu.PrefetchScalarGridSpec(
            num_scalar_prefetch=2,
            grid=(NB,),
            in_specs=[
                pl.BlockSpec((_BLK, H), lambda i, s, n: (i, 0)),
                pl.BlockSpec((_BLK, 128), lambda i, s, n: (i, 0)),
                pl.BlockSpec((1, F, H), lambda i, s, n: (s[i], 0, 0)),
                pl.BlockSpec((1, F, H), lambda i, s, n: (s[i], 0, 0)),
                pl.BlockSpec((1, F, H), lambda i, s, n: (s[i], 0, 0)),
            ],
            out_specs=pl.BlockSpec((_BLK, H), lambda i, s, n: (i, 0)),
        ),
        out_shape=jax.ShapeDtypeStruct((NPAD, H), jnp.float32),
        compiler_params=pltpu.CompilerParams(
            dimension_semantics=("arbitrary",),
        ),
    )(be, nb_used, x_sorted, ws,
      w1.astype(jnp.bfloat16), v1.astype(jnp.bfloat16),
      w2.astype(jnp.bfloat16))

    combine = pl.kernel(
        functools.partial(_combine_body, T, H),
        out_type=jax.ShapeDtypeStruct((T, H), jnp.float32),
        mesh=_mesh(),
        compiler_params=pltpu.CompilerParams(needs_layout_passes=False),
        scratch_types=[
            pltpu.VMEM((2, tt), jnp.int32),
            pltpu.VMEM((2, _LANES, H), jnp.float32),
            pltpu.VMEM((2, _LANES, H), jnp.float32),
            pltpu.VMEM((_LANES, H), jnp.float32),
            pltpu.SemaphoreType.DMA,
            pltpu.SemaphoreType.DMA,
            pltpu.SemaphoreType.DMA,
        ],
    )
    out = combine(y_sorted, pos)

    return jnp.transpose(out.reshape(S, B, H), (1, 0, 2))


# speedup vs baseline: 3.0914x; 3.0914x over previous
import jax
import jax.numpy as jnp
from jax.experimental import pallas as pl
from jax.experimental.pallas import tpu as pltpu


def _fused_moe_body(x_ref, rw_ref, w1_ref, v1_ref, w2_ref, out_ref,
                    idx2_ref, wv2_ref):
    e = pl.program_id(0)
    E = pl.num_programs(0)

    @pl.when(e == 0)
    def _route():
        x = x_ref[...]
        logits = jax.lax.dot_general(x, rw_ref[...], (((1,), (1,)), ((), ())))
        m = jnp.max(logits, axis=-1, keepdims=True)
        s = jnp.exp(logits - m)
        p = s / jnp.sum(s, axis=-1, keepdims=True)
        lane = jax.lax.broadcasted_iota(jnp.int32, p.shape, 1)
        m1 = jnp.max(p, axis=-1, keepdims=True)
        i1 = jnp.min(jnp.where(p == m1, lane, E), axis=-1, keepdims=True)
        p2 = jnp.where(lane == i1, -jnp.inf, p)
        m2 = jnp.max(p2, axis=-1, keepdims=True)
        i2 = jnp.min(jnp.where(p2 == m2, lane, E), axis=-1, keepdims=True)
        denom = m1 + m2
        idx2_ref[:, 0:1] = i1
        idx2_ref[:, 1:2] = i2
        wv2_ref[:, 0:1] = m1 / denom
        wv2_ref[:, 1:2] = m2 / denom
        out_ref[...] = jnp.zeros_like(out_ref)

    dn = (((1,), (1,)), ((), ()))
    w1b = w1_ref[0].astype(jnp.bfloat16)
    v1b = v1_ref[0].astype(jnp.bfloat16)
    w2b = w2_ref[0].astype(jnp.bfloat16)
    T = out_ref.shape[0]
    TC = 512
    for c in range(T // TC):
        sl = pl.ds(c * TC, TC)
        xc = x_ref[sl, :].astype(jnp.bfloat16)
        w_e = (jnp.where(idx2_ref[sl, 0:1] == e, wv2_ref[sl, 0:1], 0.0)
               + jnp.where(idx2_ref[sl, 1:2] == e, wv2_ref[sl, 1:2], 0.0))
        h1 = jax.lax.dot_general(xc, w1b, dn,
                                 preferred_element_type=jnp.float32)
        h2 = jax.lax.dot_general(xc, v1b, dn,
                                 preferred_element_type=jnp.float32)
        g = ((h1 * jax.nn.sigmoid(h1)) * h2).astype(jnp.bfloat16)
        y = jax.lax.dot_general(g, w2b, (((1,), (0,)), ((), ())),
                                preferred_element_type=jnp.float32)
        out_ref[sl, :] += w_e * y


def kernel(hidden_states, router_w, w1, v1, w2):
    B, S, H = hidden_states.shape
    E, F, _ = w1.shape
    x = jnp.transpose(hidden_states, (1, 0, 2)).reshape(-1, H)
    T = x.shape[0]

    out = pl.pallas_call(
        _fused_moe_body,
        grid=(E,),
        in_specs=[
            pl.BlockSpec((T, H), lambda e: (0, 0)),
            pl.BlockSpec((E, H), lambda e: (0, 0)),
            pl.BlockSpec((1, F, H), lambda e: (e, 0, 0)),
            pl.BlockSpec((1, F, H), lambda e: (e, 0, 0)),
            pl.BlockSpec((1, F, H), lambda e: (e, 0, 0)),
        ],
        out_specs=pl.BlockSpec((T, H), lambda e: (0, 0)),
        out_shape=jax.ShapeDtypeStruct((T, H), jnp.float32),
        scratch_shapes=[
            pltpu.VMEM((T, 2), jnp.int32),
            pltpu.VMEM((T, 2), jnp.float32),
        ],
        compiler_params=pltpu.CompilerParams(
            dimension_semantics=("arbitrary",),
        ),
    )(x, router_w, w1, v1, w2)

    return jnp.transpose(out.reshape(S, B, H), (1, 0, 2))


# speedup vs baseline: 3.1373x; 1.0148x over previous
import jax
import jax.numpy as jnp
from jax.experimental import pallas as pl
from jax.experimental.pallas import tpu as pltpu


def _fused_moe_body(x_ref, rw_ref, w1_ref, v1_ref, w2_ref, out_ref,
                    idx2_ref, wv2_ref):
    e = pl.program_id(0)
    E = pl.num_programs(0)

    @pl.when(e == 0)
    def _route():
        x = x_ref[...]
        logits = jax.lax.dot_general(x, rw_ref[...], (((1,), (1,)), ((), ())))
        m = jnp.max(logits, axis=-1, keepdims=True)
        s = jnp.exp(logits - m)
        p = s / jnp.sum(s, axis=-1, keepdims=True)
        lane = jax.lax.broadcasted_iota(jnp.int32, p.shape, 1)
        m1 = jnp.max(p, axis=-1, keepdims=True)
        i1 = jnp.min(jnp.where(p == m1, lane, E), axis=-1, keepdims=True)
        p2 = jnp.where(lane == i1, -jnp.inf, p)
        m2 = jnp.max(p2, axis=-1, keepdims=True)
        i2 = jnp.min(jnp.where(p2 == m2, lane, E), axis=-1, keepdims=True)
        denom = m1 + m2
        idx2_ref[:, 0:1] = i1
        idx2_ref[:, 1:2] = i2
        wv2_ref[:, 0:1] = m1 / denom
        wv2_ref[:, 1:2] = m2 / denom
        out_ref[...] = jnp.zeros_like(out_ref)

    dn = (((1,), (1,)), ((), ()))
    w1b = w1_ref[0].astype(jnp.bfloat16)
    v1b = v1_ref[0].astype(jnp.bfloat16)
    w2b = w2_ref[0].astype(jnp.bfloat16)
    T = out_ref.shape[0]
    TC = 1024
    for c in range(T // TC):
        sl = pl.ds(c * TC, TC)
        xc = x_ref[sl, :].astype(jnp.bfloat16)
        w_e = (jnp.where(idx2_ref[sl, 0:1] == e, wv2_ref[sl, 0:1], 0.0)
               + jnp.where(idx2_ref[sl, 1:2] == e, wv2_ref[sl, 1:2], 0.0))
        h1 = jax.lax.dot_general(xc, w1b, dn,
                                 preferred_element_type=jnp.float32)
        h2 = jax.lax.dot_general(xc, v1b, dn,
                                 preferred_element_type=jnp.float32)
        g = ((h1 * jax.nn.sigmoid(h1)) * h2).astype(jnp.bfloat16)
        y = jax.lax.dot_general(g, w2b, (((1,), (0,)), ((), ())),
                                preferred_element_type=jnp.float32)
        out_ref[sl, :] += w_e * y


def kernel(hidden_states, router_w, w1, v1, w2):
    B, S, H = hidden_states.shape
    E, F, _ = w1.shape
    x = jnp.transpose(hidden_states, (1, 0, 2)).reshape(-1, H)
    T = x.shape[0]

    out = pl.pallas_call(
        _fused_moe_body,
        grid=(E,),
        in_specs=[
            pl.BlockSpec((T, H), lambda e: (0, 0)),
            pl.BlockSpec((E, H), lambda e: (0, 0)),
            pl.BlockSpec((1, F, H), lambda e: (e, 0, 0)),
            pl.BlockSpec((1, F, H), lambda e: (e, 0, 0)),
            pl.BlockSpec((1, F, H), lambda e: (e, 0, 0)),
        ],
        out_specs=pl.BlockSpec((T, H), lambda e: (0, 0)),
        out_shape=jax.ShapeDtypeStruct((T, H), jnp.float32),
        scratch_shapes=[
            pltpu.VMEM((T, 2), jnp.int32),
            pltpu.VMEM((T, 2), jnp.float32),
        ],
        compiler_params=pltpu.CompilerParams(
            dimension_semantics=("arbitrary",),
        ),
    )(x, router_w, w1, v1, w2)

    return jnp.transpose(out.reshape(S, B, H), (1, 0, 2))
